# speedup vs baseline: 2.6315x; 2.6315x over previous
import jax
import jax.numpy as jnp
from jax import lax
from jax.experimental import pallas as pl
from jax.experimental.pallas import tpu as pltpu
from jax.experimental.pallas import tpu_sc as plsc

NODE_NUM = 1000000
N_UPDATES = 4194304

LANES = 128
ROWS = N_UPDATES // LANES
N_TILES = 16
ROWS_PER_TILE = ROWS // N_TILES
GROUP_ROWS = 16
N_GROUPS = ROWS_PER_TILE // GROUP_ROWS

HALF = NODE_NUM // 2
ACC = 512000
TRASH = HALF

TILE_OUT = 31264
TILE_OUT_LAST = HALF - 15 * TILE_OUT
CHUNK = 2048
FULL_CHUNKS = 15
TAIL0 = TILE_OUT - FULL_CHUNKS * CHUNK
TAIL15 = TILE_OUT_LAST - FULL_CHUNKS * CHUNK
ZBUF = 4000


def _body(idx_hbm, val_hbm, str_hbm, out_hbm,
          sums, cnts, idx_raw, idx2, vals, ones_r, zbuf, sv, cv, stv, ov):
  c = lax.axis_index("c")
  t = lax.axis_index("s")

  @pl.loop(0, ZBUF // 16)
  def _(i):
    zbuf[pl.ds(i * 16, 16)] = jnp.zeros((16,), jnp.float32)

  @pl.loop(0, LANES // 16)
  def _(i):
    ones_r[pl.ds(i * 16, 16)] = jnp.ones((16,), jnp.float32)

  per_tile = ACC // N_TILES

  @pl.loop(0, per_tile // ZBUF)
  def _(i):
    off = t * per_tile + i * ZBUF
    pltpu.sync_copy(zbuf, sums.at[pl.ds(off, ZBUF)])
    pltpu.sync_copy(zbuf, cnts.at[pl.ds(off, ZBUF)])

  plsc.subcore_barrier()

  base = c * HALF

  @pl.loop(0, N_GROUPS)
  def _(g):
    row = t * ROWS_PER_TILE + g * GROUP_ROWS
    pltpu.sync_copy(idx_hbm.at[pl.ds(row, GROUP_ROWS)], idx_raw)
    pltpu.sync_copy(val_hbm.at[pl.ds(row, GROUP_ROWS)], vals)

    @pl.loop(0, GROUP_ROWS)
    def _(j):
      src_row = idx_raw.at[j]
      dst_row = idx2.at[j]
      for b in range(LANES // 16):
        v = src_row[pl.ds(b * 16, 16)]
        loc = v - base
        m = loc.astype(jnp.uint32) < jnp.uint32(HALF)
        dst_row[pl.ds(b * 16, 16)] = jnp.where(m, loc, TRASH)
      pltpu.sync_copy(vals.at[j], sums.at[dst_row], add=True)
      pltpu.sync_copy(ones_r, cnts.at[dst_row], add=True)

  plsc.subcore_barrier()

  lbase = t * TILE_OUT
  abase = c * HALF + lbase

  def compute(off, size):
    pltpu.sync_copy(sums.at[pl.ds(lbase + off, size)], sv.at[pl.ds(0, size)])
    pltpu.sync_copy(cnts.at[pl.ds(lbase + off, size)], cv.at[pl.ds(0, size)])
    pltpu.sync_copy(str_hbm.at[pl.ds(abase + off, size)], stv.at[pl.ds(0, size)])
    for i in range(size // 16):
      s = sv[pl.ds(i * 16, 16)]
      n = cv[pl.ds(i * 16, 16)]
      z = stv[pl.ds(i * 16, 16)]
      ov[pl.ds(i * 16, 16)] = z + s / jnp.maximum(n, 1.0)
    pltpu.sync_copy(ov.at[pl.ds(0, size)], out_hbm.at[pl.ds(abase + off, size)])

  @pl.loop(0, FULL_CHUNKS)
  def _(g):
    compute(g * CHUNK, CHUNK)

  @pl.when(t < N_TILES - 1)
  def _():
    compute(FULL_CHUNKS * CHUNK, TAIL0)

  @pl.when(t == N_TILES - 1)
  def _():
    compute(FULL_CHUNKS * CHUNK, TAIL15)


@jax.jit
def kernel(node_strengths, node_indices, strengths):
  idx2d = node_indices.reshape(ROWS, LANES)
  val2d = node_strengths.reshape(ROWS, LANES)
  run = pl.kernel(
      _body,
      out_type=jax.ShapeDtypeStruct((NODE_NUM,), jnp.float32),
      mesh=plsc.VectorSubcoreMesh(core_axis_name="c", subcore_axis_name="s"),
      scratch_types=[
          pltpu.VMEM_SHARED((ACC,), jnp.float32),
          pltpu.VMEM_SHARED((ACC,), jnp.float32),
          pltpu.VMEM((GROUP_ROWS, LANES), jnp.int32),
          pltpu.VMEM((GROUP_ROWS, LANES), jnp.int32),
          pltpu.VMEM((GROUP_ROWS, LANES), jnp.float32),
          pltpu.VMEM((LANES,), jnp.float32),
          pltpu.VMEM((ZBUF,), jnp.float32),
          pltpu.VMEM((CHUNK,), jnp.float32),
          pltpu.VMEM((CHUNK,), jnp.float32),
          pltpu.VMEM((CHUNK,), jnp.float32),
          pltpu.VMEM((CHUNK,), jnp.float32),
      ],
  )
  return run(idx2d, val2d, strengths)


# speedup vs baseline: 2.6334x; 1.0007x over previous
import jax
import jax.numpy as jnp
from jax import lax
from jax.experimental import pallas as pl
from jax.experimental.pallas import tpu as pltpu
from jax.experimental.pallas import tpu_sc as plsc

NODE_NUM = 1000000
N_UPDATES = 4194304

LANES = 128
ROWS = N_UPDATES // LANES
N_TILES = 16
ROWS_PER_TILE = ROWS // N_TILES
GROUP_ROWS = 32
N_GROUPS = ROWS_PER_TILE // GROUP_ROWS

HALF = NODE_NUM // 2
ACC = 512000
TRASH = HALF

TILE_OUT = 31264
TILE_OUT_LAST = HALF - 15 * TILE_OUT
CHUNK = 2048
FULL_CHUNKS = 15
TAIL0 = TILE_OUT - FULL_CHUNKS * CHUNK
TAIL15 = TILE_OUT_LAST - FULL_CHUNKS * CHUNK
ZBUF = 4000


def _body(idx_hbm, val_hbm, str_hbm, out_hbm,
          sums, cnts, idx_raw, idx2, vals, ones_r, zbuf, sv, cv, stv, ov,
          sem_in, sem_sc):
  c = lax.axis_index("c")
  t = lax.axis_index("s")

  @pl.loop(0, ZBUF // 16)
  def _(i):
    zbuf[pl.ds(i * 16, 16)] = jnp.zeros((16,), jnp.float32)

  @pl.loop(0, GROUP_ROWS)
  def _(j):
    row = ones_r.at[j]
    for b in range(LANES // 16):
      row[pl.ds(b * 16, 16)] = jnp.ones((16,), jnp.float32)

  per_tile = ACC // N_TILES

  @pl.loop(0, per_tile // ZBUF)
  def _(i):
    off = t * per_tile + i * ZBUF
    pltpu.sync_copy(zbuf, sums.at[pl.ds(off, ZBUF)])
    pltpu.sync_copy(zbuf, cnts.at[pl.ds(off, ZBUF)])

  plsc.subcore_barrier()

  base = c * HALF
  LAG = 8

  def sc_row_descs(j):
    return (
        pltpu.make_async_copy(vals.at[j], sums.at[idx2.at[j]], sem_sc),
        pltpu.make_async_copy(ones_r.at[j], cnts.at[idx2.at[j]], sem_sc),
    )

  @pl.loop(0, N_GROUPS)
  def _(g):
    row = t * ROWS_PER_TILE + g * GROUP_ROWS
    pltpu.sync_copy(idx_hbm.at[pl.ds(row, GROUP_ROWS)], idx_raw)
    pltpu.sync_copy(val_hbm.at[pl.ds(row, GROUP_ROWS)], vals)

    @pl.loop(0, GROUP_ROWS)
    def _(j):
      src_row = idx_raw.at[j]
      dst_row = idx2.at[j]
      for b in range(LANES // 16):
        v = src_row[pl.ds(b * 16, 16)]
        loc = v - base
        m = loc.astype(jnp.uint32) < jnp.uint32(HALF)
        dst_row[pl.ds(b * 16, 16)] = jnp.where(m, loc, TRASH)

    @pl.loop(0, GROUP_ROWS)
    def _(j):
      pltpu.sync_copy(vals.at[j], sums.at[idx2.at[j]], add=True)
      pltpu.sync_copy(ones_r.at[j], cnts.at[idx2.at[j]], add=True)

  plsc.subcore_barrier()

  lbase = t * TILE_OUT
  abase = c * HALF + lbase

  def compute(off, size):
    pltpu.sync_copy(sums.at[pl.ds(lbase + off, size)], sv.at[pl.ds(0, size)])
    pltpu.sync_copy(cnts.at[pl.ds(lbase + off, size)], cv.at[pl.ds(0, size)])
    pltpu.sync_copy(str_hbm.at[pl.ds(abase + off, size)], stv.at[pl.ds(0, size)])
    for i in range(size // 16):
      s = sv[pl.ds(i * 16, 16)]
      n = cv[pl.ds(i * 16, 16)]
      z = stv[pl.ds(i * 16, 16)]
      ov[pl.ds(i * 16, 16)] = z + s / jnp.maximum(n, 1.0)
    pltpu.sync_copy(ov.at[pl.ds(0, size)], out_hbm.at[pl.ds(abase + off, size)])

  @pl.loop(0, FULL_CHUNKS)
  def _(g):
    compute(g * CHUNK, CHUNK)

  @pl.when(t < N_TILES - 1)
  def _():
    compute(FULL_CHUNKS * CHUNK, TAIL0)

  @pl.when(t == N_TILES - 1)
  def _():
    compute(FULL_CHUNKS * CHUNK, TAIL15)


@jax.jit
def kernel(node_strengths, node_indices, strengths):
  idx2d = node_indices.reshape(ROWS, LANES)
  val2d = node_strengths.reshape(ROWS, LANES)
  run = pl.kernel(
      _body,
      out_type=jax.ShapeDtypeStruct((NODE_NUM,), jnp.float32),
      mesh=plsc.VectorSubcoreMesh(core_axis_name="c", subcore_axis_name="s"),
      scratch_types=[
          pltpu.VMEM_SHARED((ACC,), jnp.float32),
          pltpu.VMEM_SHARED((ACC,), jnp.float32),
          pltpu.VMEM((GROUP_ROWS, LANES), jnp.int32),
          pltpu.VMEM((GROUP_ROWS, LANES), jnp.int32),
          pltpu.VMEM((GROUP_ROWS, LANES), jnp.float32),
          pltpu.VMEM((GROUP_ROWS, LANES), jnp.float32),
          pltpu.VMEM((ZBUF,), jnp.float32),
          pltpu.VMEM((CHUNK,), jnp.float32),
          pltpu.VMEM((CHUNK,), jnp.float32),
          pltpu.VMEM((CHUNK,), jnp.float32),
          pltpu.VMEM((CHUNK,), jnp.float32),
          pltpu.SemaphoreType.DMA,
          pltpu.SemaphoreType.DMA,
      ],
  )
  return run(idx2d, val2d, strengths)
